# baseline (speedup 1.0000x reference)
import functools

import jax
import jax.numpy as jnp
from jax import lax
from jax.experimental import pallas as pl
from jax.experimental.pallas import tpu as pltpu
from jax.experimental.pallas import tpu_sc as plsc

BATCH = 16384
VOCAB = 100000
DIM = 16
NTAB = 26
NC = 2
NS = 16
L = 16
NW = NC * NS
BPW = BATCH // NW

BLKV = 5120
NBLK = 20
VPAD = NBLK * BLKV


def _scores_body(*refs):
    w_ref = refs[0]
    b_ref = refs[1]
    beta_refs = refs[2:2 + 13]
    tab_refs = refs[15:15 + NTAB]
    out_refs = refs[15 + NTAB:15 + 2 * NTAB]
    const_ref = refs[15 + 2 * NTAB]
    for j in range(NTAB):
        wj = w_ref[pl.ds(13 + j * DIM, DIM), :]
        s = lax.dot_general(wj, tab_refs[j][...],
                            (((0,), (0,)), ((), ())),
                            preferred_element_type=jnp.float32)
        out_refs[j][...] = s.reshape(BLKV)
    const = b_ref[0]
    for i in range(13):
        const = const + beta_refs[i][0] * w_ref[i, 0]
    const_ref[...] = jnp.full((L,), 0.0, jnp.float32) + const


def _make_scores_call():
    in_specs = [pl.BlockSpec((429, 1), lambda i: (0, 0)),
                pl.BlockSpec((1,), lambda i: (0,))]
    in_specs += [pl.BlockSpec((1,), lambda i: (0,)) for _ in range(13)]
    in_specs += [pl.BlockSpec((DIM, BLKV), lambda i: (0, i))
                 for _ in range(NTAB)]
    out_specs = [pl.BlockSpec((BLKV,), lambda i: (i,)) for _ in range(NTAB)]
    out_specs += [pl.BlockSpec((L,), lambda i: (0,))]
    out_shape = [jax.ShapeDtypeStruct((VPAD,), jnp.float32)
                 for _ in range(NTAB)]
    out_shape += [jax.ShapeDtypeStruct((L,), jnp.float32)]
    return pl.pallas_call(
        _scores_body,
        grid=(NBLK,),
        in_specs=in_specs,
        out_specs=out_specs,
        out_shape=out_shape,
    )


_scores_tc = _make_scores_call()


def _gather_body(*refs):
    idx_refs = refs[0:NTAB]
    const_hbm = refs[NTAB]
    score_refs = refs[NTAB + 1:2 * NTAB + 1]
    out_hbm = refs[2 * NTAB + 1]
    (idx_v, vals_v, const_v, out_v, sem, sem2) = refs[2 * NTAB + 2:]

    wid = lax.axis_index("s") * NC + lax.axis_index("c")
    base = wid * BPW

    pltpu.sync_copy(const_hbm, const_v)
    idx_copies = [
        pltpu.async_copy(idx_refs[j].at[pl.ds(base, BPW)], idx_v.at[j], sem)
        for j in range(NTAB)
    ]
    gathers = []
    for j in range(NTAB):
        idx_copies[j].wait()
        gathers.append(
            pltpu.async_copy(score_refs[j].at[idx_v.at[j]], vals_v.at[j],
                             sem2))
    const = const_v[...]
    for cp in gathers:
        cp.wait()

    @pl.loop(0, BPW // L)
    def _(k):
        off = pl.multiple_of(k * L, L)
        acc = vals_v[0, pl.ds(off, L)] + const
        for j in range(1, NTAB):
            acc = acc + vals_v[j, pl.ds(off, L)]
        out_v[pl.ds(off, L)] = 1.0 / (1.0 + jnp.exp(-acc))

    pltpu.sync_copy(out_v, out_hbm.at[pl.ds(base, BPW)])


_gather_sc = functools.partial(
    pl.kernel,
    out_type=jax.ShapeDtypeStruct((BATCH,), jnp.float32),
    mesh=plsc.VectorSubcoreMesh(
        core_axis_name="c", subcore_axis_name="s",
        num_cores=NC, num_subcores=NS,
    ),
    compiler_params=pltpu.CompilerParams(
        needs_layout_passes=False, use_tc_tiling_on_sc=False
    ),
    scratch_types=[
        pltpu.VMEM((NTAB, BPW), jnp.int32),
        pltpu.VMEM((NTAB, BPW), jnp.float32),
        pltpu.VMEM((L,), jnp.float32),
        pltpu.VMEM((BPW,), jnp.float32),
        pltpu.SemaphoreType.DMA,
        pltpu.SemaphoreType.DMA,
    ],
)(_gather_body)


def kernel(I1, I2, I3, I4, I5, I6, I7, I8, I9, I10, I11, I12, I13, ln_gamma_I1, ln_gamma_I2, ln_gamma_I3, ln_gamma_I4, ln_gamma_I5, ln_gamma_I6, ln_gamma_I7, ln_gamma_I8, ln_gamma_I9, ln_gamma_I10, ln_gamma_I11, ln_gamma_I12, ln_gamma_I13, ln_beta_I1, ln_beta_I2, ln_beta_I3, ln_beta_I4, ln_beta_I5, ln_beta_I6, ln_beta_I7, ln_beta_I8, ln_beta_I9, ln_beta_I10, ln_beta_I11, ln_beta_I12, ln_beta_I13, C1, C2, C3, C4, C5, C6, C7, C8, C9, C10, C11, C12, C13, C14, C15, C16, C17, C18, C19, C20, C21, C22, C23, C24, C25, C26, emb_C1, emb_C2, emb_C3, emb_C4, emb_C5, emb_C6, emb_C7, emb_C8, emb_C9, emb_C10, emb_C11, emb_C12, emb_C13, emb_C14, emb_C15, emb_C16, emb_C17, emb_C18, emb_C19, emb_C20, emb_C21, emb_C22, emb_C23, emb_C24, emb_C25, emb_C26, W, b):
    Cs = [C1, C2, C3, C4, C5, C6, C7, C8, C9, C10, C11, C12, C13, C14,
          C15, C16, C17, C18, C19, C20, C21, C22, C23, C24, C25, C26]
    tabs = [emb_C1, emb_C2, emb_C3, emb_C4, emb_C5, emb_C6, emb_C7, emb_C8,
            emb_C9, emb_C10, emb_C11, emb_C12, emb_C13, emb_C14, emb_C15,
            emb_C16, emb_C17, emb_C18, emb_C19, emb_C20, emb_C21, emb_C22,
            emb_C23, emb_C24, emb_C25, emb_C26]
    betas = [ln_beta_I1, ln_beta_I2, ln_beta_I3, ln_beta_I4, ln_beta_I5,
             ln_beta_I6, ln_beta_I7, ln_beta_I8, ln_beta_I9, ln_beta_I10,
             ln_beta_I11, ln_beta_I12, ln_beta_I13]

    idxs = [c.reshape(BATCH) for c in Cs]

    outs = _scores_tc(W, b, *betas, *(t.T for t in tabs))
    scores, const16 = outs[:NTAB], outs[NTAB]
    out = _gather_sc(*idxs, const16, *scores)
    return out.reshape(BATCH, 1)


# speedup vs baseline: 1.0027x; 1.0027x over previous
import functools

import jax
import jax.numpy as jnp
from jax import lax
from jax.experimental import pallas as pl
from jax.experimental.pallas import tpu as pltpu
from jax.experimental.pallas import tpu_sc as plsc

BATCH = 16384
VOCAB = 100000
DIM = 16
NTAB = 26
NC = 2
NS = 16
L = 16
NW = NC * NS
BPW = BATCH // NW

BLKV = 5120
NBLK = 20
VPAD = NBLK * BLKV


def _scores_body(*refs):
    w_ref = refs[0]
    b_ref = refs[1]
    beta_refs = refs[2:2 + 13]
    tab_refs = refs[15:15 + NTAB]
    out_refs = refs[15 + NTAB:15 + 2 * NTAB]
    const_ref = refs[15 + 2 * NTAB]
    for j in range(NTAB):
        wj = w_ref[pl.ds(13 + j * DIM, DIM), :]
        s = lax.dot_general(wj, tab_refs[j][...],
                            (((0,), (0,)), ((), ())),
                            preferred_element_type=jnp.float32)
        out_refs[j][...] = s.reshape(BLKV)
    const = b_ref[0]
    for i in range(13):
        const = const + beta_refs[i][0] * w_ref[i, 0]
    const_ref[...] = jnp.full((L,), 0.0, jnp.float32) + const


def _make_scores_call():
    in_specs = [pl.BlockSpec((429, 1), lambda i: (0, 0)),
                pl.BlockSpec((1,), lambda i: (0,))]
    in_specs += [pl.BlockSpec((1,), lambda i: (0,)) for _ in range(13)]
    in_specs += [pl.BlockSpec((DIM, BLKV), lambda i: (0, i))
                 for _ in range(NTAB)]
    out_specs = [pl.BlockSpec((BLKV,), lambda i: (i,)) for _ in range(NTAB)]
    out_specs += [pl.BlockSpec((L,), lambda i: (0,))]
    out_shape = [jax.ShapeDtypeStruct((VPAD,), jnp.float32)
                 for _ in range(NTAB)]
    out_shape += [jax.ShapeDtypeStruct((L,), jnp.float32)]
    return pl.pallas_call(
        _scores_body,
        grid=(NBLK,),
        in_specs=in_specs,
        out_specs=out_specs,
        out_shape=out_shape,
    )


_scores_tc = _make_scores_call()


def _gather_body(*refs):
    idx_refs = refs[0:NTAB]
    const_hbm = refs[NTAB]
    score_refs = refs[NTAB + 1:2 * NTAB + 1]
    out_hbm = refs[2 * NTAB + 1]
    (idx_v, vals_v, const_v, acc_v, out_v, sem, sem2) = refs[2 * NTAB + 2:]

    wid = lax.axis_index("s") * NC + lax.axis_index("c")
    base = wid * BPW

    pltpu.sync_copy(const_hbm, const_v)
    idx_copies = [
        pltpu.async_copy(idx_refs[j].at[pl.ds(base, BPW)], idx_v.at[j], sem)
        for j in range(NTAB)
    ]
    gathers = []
    for j in range(NTAB):
        idx_copies[j].wait()
        gathers.append(
            pltpu.async_copy(score_refs[j].at[idx_v.at[j]], vals_v.at[j],
                             sem2))
    const = const_v[...]
    half = NTAB // 2
    for cp in gathers[:half]:
        cp.wait()

    @pl.loop(0, BPW // L)
    def _(k):
        off = pl.multiple_of(k * L, L)
        acc = vals_v[0, pl.ds(off, L)] + const
        for j in range(1, half):
            acc = acc + vals_v[j, pl.ds(off, L)]
        acc_v[pl.ds(off, L)] = acc

    for cp in gathers[half:]:
        cp.wait()

    @pl.loop(0, BPW // L)
    def _(k):
        off = pl.multiple_of(k * L, L)
        acc = acc_v[pl.ds(off, L)]
        for j in range(half, NTAB):
            acc = acc + vals_v[j, pl.ds(off, L)]
        out_v[pl.ds(off, L)] = 1.0 / (1.0 + jnp.exp(-acc))

    pltpu.sync_copy(out_v, out_hbm.at[pl.ds(base, BPW)])


_gather_sc = functools.partial(
    pl.kernel,
    out_type=jax.ShapeDtypeStruct((BATCH,), jnp.float32),
    mesh=plsc.VectorSubcoreMesh(
        core_axis_name="c", subcore_axis_name="s",
        num_cores=NC, num_subcores=NS,
    ),
    compiler_params=pltpu.CompilerParams(
        needs_layout_passes=False, use_tc_tiling_on_sc=False
    ),
    scratch_types=[
        pltpu.VMEM((NTAB, BPW), jnp.int32),
        pltpu.VMEM((NTAB, BPW), jnp.float32),
        pltpu.VMEM((L,), jnp.float32),
        pltpu.VMEM((BPW,), jnp.float32),
        pltpu.VMEM((BPW,), jnp.float32),
        pltpu.SemaphoreType.DMA,
        pltpu.SemaphoreType.DMA,
    ],
)(_gather_body)


def kernel(I1, I2, I3, I4, I5, I6, I7, I8, I9, I10, I11, I12, I13, ln_gamma_I1, ln_gamma_I2, ln_gamma_I3, ln_gamma_I4, ln_gamma_I5, ln_gamma_I6, ln_gamma_I7, ln_gamma_I8, ln_gamma_I9, ln_gamma_I10, ln_gamma_I11, ln_gamma_I12, ln_gamma_I13, ln_beta_I1, ln_beta_I2, ln_beta_I3, ln_beta_I4, ln_beta_I5, ln_beta_I6, ln_beta_I7, ln_beta_I8, ln_beta_I9, ln_beta_I10, ln_beta_I11, ln_beta_I12, ln_beta_I13, C1, C2, C3, C4, C5, C6, C7, C8, C9, C10, C11, C12, C13, C14, C15, C16, C17, C18, C19, C20, C21, C22, C23, C24, C25, C26, emb_C1, emb_C2, emb_C3, emb_C4, emb_C5, emb_C6, emb_C7, emb_C8, emb_C9, emb_C10, emb_C11, emb_C12, emb_C13, emb_C14, emb_C15, emb_C16, emb_C17, emb_C18, emb_C19, emb_C20, emb_C21, emb_C22, emb_C23, emb_C24, emb_C25, emb_C26, W, b):
    Cs = [C1, C2, C3, C4, C5, C6, C7, C8, C9, C10, C11, C12, C13, C14,
          C15, C16, C17, C18, C19, C20, C21, C22, C23, C24, C25, C26]
    tabs = [emb_C1, emb_C2, emb_C3, emb_C4, emb_C5, emb_C6, emb_C7, emb_C8,
            emb_C9, emb_C10, emb_C11, emb_C12, emb_C13, emb_C14, emb_C15,
            emb_C16, emb_C17, emb_C18, emb_C19, emb_C20, emb_C21, emb_C22,
            emb_C23, emb_C24, emb_C25, emb_C26]
    betas = [ln_beta_I1, ln_beta_I2, ln_beta_I3, ln_beta_I4, ln_beta_I5,
             ln_beta_I6, ln_beta_I7, ln_beta_I8, ln_beta_I9, ln_beta_I10,
             ln_beta_I11, ln_beta_I12, ln_beta_I13]

    idxs = [c.reshape(BATCH) for c in Cs]

    outs = _scores_tc(W, b, *betas, *(t.T for t in tabs))
    scores, const16 = outs[:NTAB], outs[NTAB]
    out = _gather_sc(*idxs, const16, *scores)
    return out.reshape(BATCH, 1)
